# baseline (speedup 1.0000x reference)
import functools

import jax
import jax.numpy as jnp
from jax import lax
from jax.experimental import pallas as pl
from jax.experimental.pallas import tpu as pltpu
from jax.experimental.pallas import tpu_sc as plsc

NC, NS, L = 2, 16, 16
NW = NC * NS
B, SEQ = 4096, 200
WD, PD, OD = 64, 16, 96
B_PER_W = B // NW
NSEQ = 4
NCHUNK = B_PER_W // NSEQ
SPLITS = ((0, 96), (96, 104))


@functools.partial(
    pl.kernel,
    out_type=jax.ShapeDtypeStruct((B, SEQ, OD), jnp.float32),
    mesh=plsc.VectorSubcoreMesh(core_axis_name="c", subcore_axis_name="s"),
    scratch_types=[
        pltpu.VMEM((NSEQ, SEQ), jnp.int32),
        pltpu.VMEM((NSEQ, SEQ), jnp.int32),
        pltpu.VMEM((NSEQ, SEQ), jnp.int32),
        pltpu.VMEM((NSEQ, SEQ, WD), jnp.float32),
        pltpu.VMEM((NSEQ, SEQ, PD), jnp.float32),
        pltpu.VMEM((NSEQ, SEQ, PD), jnp.float32),
        pltpu.SemaphoreType.DMA,
    ],
    compiler_params=pltpu.CompilerParams(use_tc_tiling_on_sc=False,
                                         needs_layout_passes=False),
)
def _embed_sc(words_hbm, head_hbm, tail_hbm, wt_hbm, ht_hbm, tt_hbm,
              out_hbm, widx_v, hidx_v, tidx_v, wrow_v, hrow_v, trow_v, sem):
    wid = lax.axis_index("s") * NC + lax.axis_index("c")
    seq0 = wid * B_PER_W

    def chunk_body(ci, _):
        b0 = seq0 + ci * NSEQ

        pltpu.sync_copy(words_hbm.at[pl.ds(b0, NSEQ)], widx_v)
        pltpu.sync_copy(head_hbm.at[pl.ds(b0, NSEQ)], hidx_v)
        pltpu.sync_copy(tail_hbm.at[pl.ds(b0, NSEQ)], tidx_v)

        copies = []
        for i in range(NSEQ):
            for off, width in SPLITS:
                sl = pl.ds(off, width)
                copies.append(pltpu.async_copy(
                    wt_hbm.at[widx_v.at[i, sl]], wrow_v.at[i, sl], sem))
                copies.append(pltpu.async_copy(
                    ht_hbm.at[hidx_v.at[i, sl]], hrow_v.at[i, sl], sem))
                copies.append(pltpu.async_copy(
                    tt_hbm.at[tidx_v.at[i, sl]], trow_v.at[i, sl], sem))
        for c in copies:
            c.wait()

        def fixup_body(i, _):
            for o in list(range(0, SEQ - L, L)) + [SEQ - L]:
                idxs = widx_v[i, pl.ds(o, L)]
                msk = idxs == 0

                @pl.when(jnp.min(idxs) == 0)
                def _():
                    toks = o + lax.iota(jnp.int32, L)
                    seqv = jnp.full((L,), i, jnp.int32)
                    zf = jnp.zeros((L,), jnp.float32)
                    for col in range(WD):
                        plsc.store_scatter(
                            wrow_v,
                            [seqv, toks, jnp.full((L,), col, jnp.int32)],
                            zf, mask=msk)
            return 0

        lax.fori_loop(0, NSEQ, fixup_body, 0)

        dst = out_hbm.at[pl.ds(b0, NSEQ)]
        pltpu.sync_copy(wrow_v, dst.at[:, :, pl.ds(0, WD)])
        pltpu.sync_copy(hrow_v, dst.at[:, :, pl.ds(WD, PD)])
        pltpu.sync_copy(trow_v, dst.at[:, :, pl.ds(WD + PD, PD)])
        return 0

    lax.fori_loop(0, NCHUNK, chunk_body, 0)


def kernel(words, head_pos, tail_pos, word_table, head_pos_table, tail_pos_table):
    ht = head_pos_table.at[0].set(0.0)
    tt = tail_pos_table.at[0].set(0.0)
    return _embed_sc(words, head_pos, tail_pos, word_table, ht, tt)
